# baseline (speedup 1.0000x reference)
import functools

import jax
import jax.numpy as jnp
from jax import lax
from jax.experimental import pallas as pl
from jax.experimental.pallas import tpu as pltpu
from jax.experimental.pallas import tpu_sc as plsc

_B = 16384
_NC, _NS, _L = 1, 16, 16
_NW = _NC * _NS
_BPW = _B // _NW
_G = _BPW // _L
_D = 21
_WROWS, _DROWS, _E = 24, 42, 8

_NORM = {
    'lngs': (116.35, 0.22),
    'lats': (39.92, 0.18),
    'temperature': (12.5, 9.3),
    'dist_gap': (1.2, 0.8),
    'dist': (18.7, 10.4),
}

_mesh = plsc.VectorSubcoreMesh(core_axis_name="c", subcore_axis_name="s", num_cores=1)


@functools.partial(
    pl.kernel,
    out_type=jax.ShapeDtypeStruct((_D, _B), jnp.float32),
    mesh=_mesh,
    compiler_params=pltpu.CompilerParams(
        needs_layout_passes=False,
        skip_device_barrier=True,
        disable_bounds_checks=True,
    ),
    scratch_types=[
        pltpu.VMEM((_BPW,), jnp.float32),
        pltpu.VMEM((_BPW,), jnp.float32),
        pltpu.VMEM((_BPW,), jnp.float32),
        pltpu.VMEM((_BPW,), jnp.float32),
        pltpu.VMEM((_BPW,), jnp.float32),
        pltpu.VMEM((_BPW,), jnp.int32),
        pltpu.VMEM((_BPW,), jnp.int32),
        pltpu.VMEM((_E, _WROWS), jnp.float32),
        pltpu.VMEM((_E, _DROWS), jnp.float32),
        pltpu.VMEM((_D, _BPW), jnp.float32),
        pltpu.SemaphoreType.DMA,
    ],
)
def _net_sc(lngs_h, lats_h, w_h, d_h, temp_h, gap_h, dist_h, wt_h, dt_h,
            out_h,
            lngs_v, lats_v, temp_v, gap_v, dist_v, w_v, d_v, wt_v, dt_v,
            out_v, sem):
    wid = lax.axis_index("s") * _NC + lax.axis_index("c")
    base = wid * _BPW
    sl = pl.ds(base, _BPW)
    copies = [
        pltpu.async_copy(lngs_h.at[sl], lngs_v, sem),
        pltpu.async_copy(lats_h.at[sl], lats_v, sem),
        pltpu.async_copy(temp_h.at[sl], temp_v, sem),
        pltpu.async_copy(gap_h.at[sl], gap_v, sem),
        pltpu.async_copy(dist_h.at[sl], dist_v, sem),
        pltpu.async_copy(w_h.at[sl], w_v, sem),
        pltpu.async_copy(d_h.at[sl], d_v, sem),
        pltpu.async_copy(wt_h, wt_v, sem),
        pltpu.async_copy(dt_h, dt_v, sem),
    ]
    for c in copies:
        c.wait()

    def _norm(vec, name):
        m, s = _NORM[name]
        return (vec - jnp.float32(m)) * jnp.float32(1.0 / s)

    _GB = _G // 4
    out_copies = []
    for b in range(4):
        @plsc.parallel_loop(b * _GB, (b + 1) * _GB, 1, unroll=1)
        def body(g):
            s16 = pl.ds(g * _L, _L)
            out_v[0, s16] = _norm(lngs_v[s16], 'lngs')
            out_v[1, s16] = _norm(lats_v[s16], 'lats')
            wi = w_v[s16]
            di = d_v[s16]

            for c in range(_E):
                out_v[2 + c, s16] = plsc.load_gather(
                    wt_v, [jnp.full((_L,), c, jnp.int32), wi])
            for c in range(_E):
                out_v[10 + c, s16] = plsc.load_gather(
                    dt_v, [jnp.full((_L,), c, jnp.int32), di])
            out_v[18, s16] = _norm(temp_v[s16], 'temperature')
            out_v[19, s16] = _norm(gap_v[s16], 'dist_gap')
            out_v[20, s16] = _norm(dist_v[s16], 'dist')

        nb = _GB * _L
        out_copies.append(pltpu.async_copy(
            out_v.at[:, pl.ds(b * nb, nb)],
            out_h.at[:, pl.ds(base + b * nb, nb)], sem))
    for c in out_copies:
        c.wait()


def kernel(lngs, lats, weather, wind, temperature, dist_gap, dist,
           weather_table, wind_table):
    out_t = _net_sc(lngs, lats, weather, wind, temperature, dist_gap, dist,
                    weather_table.T, wind_table.T)
    return out_t.T


# speedup vs baseline: 1.0634x; 1.0634x over previous
import functools

import jax
import jax.numpy as jnp
from jax import lax
from jax.experimental import pallas as pl
from jax.experimental.pallas import tpu as pltpu
from jax.experimental.pallas import tpu_sc as plsc

_B = 16384
_NC, _NS, _L = 1, 16, 16
_NW = _NC * _NS
_BPW = _B // _NW
_G = _BPW // _L
_D = 21
_WROWS, _DROWS, _E = 24, 42, 8

_NORM = {
    'lngs': (116.35, 0.22),
    'lats': (39.92, 0.18),
    'temperature': (12.5, 9.3),
    'dist_gap': (1.2, 0.8),
    'dist': (18.7, 10.4),
}

_mesh = plsc.VectorSubcoreMesh(core_axis_name="c", subcore_axis_name="s", num_cores=1)


@functools.partial(
    pl.kernel,
    out_type=jax.ShapeDtypeStruct((_D, _B), jnp.float32),
    mesh=_mesh,
    compiler_params=pltpu.CompilerParams(
        needs_layout_passes=False,
        skip_device_barrier=True,
        disable_bounds_checks=True,
    ),
    scratch_types=[
        pltpu.VMEM((_BPW,), jnp.float32),
        pltpu.VMEM((_BPW,), jnp.float32),
        pltpu.VMEM((_BPW,), jnp.float32),
        pltpu.VMEM((_BPW,), jnp.float32),
        pltpu.VMEM((_BPW,), jnp.float32),
        pltpu.VMEM((_BPW,), jnp.int32),
        pltpu.VMEM((_BPW,), jnp.int32),
        pltpu.VMEM((_E, _WROWS), jnp.float32),
        pltpu.VMEM((_E, _DROWS), jnp.float32),
        pltpu.VMEM((_D, _BPW), jnp.float32),
        pltpu.SemaphoreType.DMA,
    ],
)
def _net_sc(lngs_h, lats_h, w_h, d_h, temp_h, gap_h, dist_h, wt_h, dt_h,
            out_h,
            lngs_v, lats_v, temp_v, gap_v, dist_v, w_v, d_v, wt_v, dt_v,
            out_v, sem):
    wid = lax.axis_index("s") * _NC + lax.axis_index("c")
    base = wid * _BPW
    sl = pl.ds(base, _BPW)
    copies = [
        pltpu.async_copy(lngs_h.at[sl], lngs_v, sem),
        pltpu.async_copy(lats_h.at[sl], lats_v, sem),
        pltpu.async_copy(temp_h.at[sl], temp_v, sem),
        pltpu.async_copy(gap_h.at[sl], gap_v, sem),
        pltpu.async_copy(dist_h.at[sl], dist_v, sem),
        pltpu.async_copy(w_h.at[sl], w_v, sem),
        pltpu.async_copy(d_h.at[sl], d_v, sem),
        pltpu.async_copy(wt_h, wt_v, sem),
        pltpu.async_copy(dt_h, dt_v, sem),
    ]
    for c in copies:
        c.wait()

    def _norm(vec, name):
        m, s = _NORM[name]
        return (vec - jnp.float32(m)) * jnp.float32(1.0 / s)

    @plsc.parallel_loop(0, _G, 1, unroll=2)
    def body(g):
        s16 = pl.ds(g * _L, _L)
        out_v[0, s16] = _norm(lngs_v[s16], 'lngs')
        out_v[1, s16] = _norm(lats_v[s16], 'lats')
        wi = w_v[s16]
        di = d_v[s16]

        for c in range(_E):
            out_v[2 + c, s16] = plsc.load_gather(
                wt_v, [jnp.full((_L,), c, jnp.int32), wi])
        for c in range(_E):
            out_v[10 + c, s16] = plsc.load_gather(
                dt_v, [jnp.full((_L,), c, jnp.int32), di])
        out_v[18, s16] = _norm(temp_v[s16], 'temperature')
        out_v[19, s16] = _norm(gap_v[s16], 'dist_gap')
        out_v[20, s16] = _norm(dist_v[s16], 'dist')

    pltpu.sync_copy(out_v, out_h.at[:, sl])


def kernel(lngs, lats, weather, wind, temperature, dist_gap, dist,
           weather_table, wind_table):
    out_t = _net_sc(lngs, lats, weather, wind, temperature, dist_gap, dist,
                    weather_table.T, wind_table.T)
    return out_t.T


# speedup vs baseline: 1.0783x; 1.0141x over previous
import functools

import jax
import jax.numpy as jnp
from jax import lax
from jax.experimental import pallas as pl
from jax.experimental.pallas import tpu as pltpu
from jax.experimental.pallas import tpu_sc as plsc

_B = 16384
_NC, _NS, _L = 1, 16, 16
_NW = _NC * _NS
_BPW = _B // _NW
_G = _BPW // _L
_D = 21
_WROWS, _DROWS, _E = 24, 42, 8

_NORM = {
    'lngs': (116.35, 0.22),
    'lats': (39.92, 0.18),
    'temperature': (12.5, 9.3),
    'dist_gap': (1.2, 0.8),
    'dist': (18.7, 10.4),
}

_mesh = plsc.VectorSubcoreMesh(core_axis_name="c", subcore_axis_name="s", num_cores=1)


@functools.partial(
    pl.kernel,
    out_type=jax.ShapeDtypeStruct((_D, _B), jnp.float32),
    mesh=_mesh,
    compiler_params=pltpu.CompilerParams(
        needs_layout_passes=False,
        skip_device_barrier=True,
        disable_bounds_checks=True,
    ),
    scratch_types=[
        pltpu.VMEM((_BPW,), jnp.float32),
        pltpu.VMEM((_BPW,), jnp.float32),
        pltpu.VMEM((_BPW,), jnp.float32),
        pltpu.VMEM((_BPW,), jnp.float32),
        pltpu.VMEM((_BPW,), jnp.float32),
        pltpu.VMEM((_BPW,), jnp.int32),
        pltpu.VMEM((_BPW,), jnp.int32),
        pltpu.VMEM((_E, _WROWS), jnp.float32),
        pltpu.VMEM((_E, _DROWS), jnp.float32),
        pltpu.VMEM((_D, _BPW), jnp.float32),
        pltpu.SemaphoreType.DMA,
    ],
)
def _net_sc(lngs_h, lats_h, w_h, d_h, temp_h, gap_h, dist_h, wt_h, dt_h,
            out_h,
            lngs_v, lats_v, temp_v, gap_v, dist_v, w_v, d_v, wt_v, dt_v,
            out_v, sem):
    wid = lax.axis_index("s") * _NC + lax.axis_index("c")
    base = wid * _BPW
    sl = pl.ds(base, _BPW)
    copies = [
        pltpu.async_copy(lngs_h.at[sl], lngs_v, sem),
        pltpu.async_copy(lats_h.at[sl], lats_v, sem),
        pltpu.async_copy(temp_h.at[sl], temp_v, sem),
        pltpu.async_copy(gap_h.at[sl], gap_v, sem),
        pltpu.async_copy(dist_h.at[sl], dist_v, sem),
        pltpu.async_copy(w_h.at[sl], w_v, sem),
        pltpu.async_copy(d_h.at[sl], d_v, sem),
        pltpu.async_copy(wt_h, wt_v, sem),
        pltpu.async_copy(dt_h, dt_v, sem),
    ]
    for c in copies:
        c.wait()

    def _norm(vec, name):
        m, s = _NORM[name]
        return (vec - jnp.float32(m)) * jnp.float32(1.0 / s)

    @plsc.parallel_loop(0, _G, 1, unroll=1)
    def body(g):
        s16 = pl.ds(g * _L, _L)
        out_v[0, s16] = _norm(lngs_v[s16], 'lngs')
        out_v[1, s16] = _norm(lats_v[s16], 'lats')
        wi = w_v[s16]
        di = d_v[s16]

        for c in range(_E):
            out_v[2 + c, s16] = plsc.load_gather(
                wt_v, [jnp.full((_L,), c, jnp.int32), wi])
        for c in range(_E):
            out_v[10 + c, s16] = plsc.load_gather(
                dt_v, [jnp.full((_L,), c, jnp.int32), di])
        out_v[18, s16] = _norm(temp_v[s16], 'temperature')
        out_v[19, s16] = _norm(gap_v[s16], 'dist_gap')
        out_v[20, s16] = _norm(dist_v[s16], 'dist')

    pltpu.sync_copy(out_v, out_h.at[:, sl])


def kernel(lngs, lats, weather, wind, temperature, dist_gap, dist,
           weather_table, wind_table):
    out_t = _net_sc(lngs, lats, weather, wind, temperature, dist_gap, dist,
                    weather_table.T, wind_table.T)
    return out_t.T
